# baseline (speedup 1.0000x reference)
import functools

import jax
import jax.numpy as jnp
from jax import lax
from jax.experimental import pallas as pl
from jax.experimental.pallas import tpu as pltpu

N = 10000
E = 320000
D = 128
ED = 16

_F32 = jnp.float32


def _prep_body(nf_ref, g_ref, b_ref, wa_ref, wb_ref, p_ref, q_ref):
    x = nf_ref[...]
    mu = jnp.mean(x, axis=1, keepdims=True)
    var = jnp.mean((x - mu) ** 2, axis=1, keepdims=True)
    h = (x - mu) * lax.rsqrt(var + 1e-5) * g_ref[...][None, :] + b_ref[...][None, :]
    p_ref[...] = jnp.dot(h, wa_ref[...], preferred_element_type=_F32)
    q_ref[...] = jnp.dot(h, wb_ref[...], preferred_element_type=_F32)


def _prep(nf, gamma, beta, w1a, w1b):
    blk = 1000
    grid = (N // blk,)
    return pl.pallas_call(
        _prep_body,
        grid=grid,
        in_specs=[
            pl.BlockSpec((blk, D), lambda i: (i, 0)),
            pl.BlockSpec((D,), lambda i: (0,)),
            pl.BlockSpec((D,), lambda i: (0,)),
            pl.BlockSpec((D, D), lambda i: (0, 0)),
            pl.BlockSpec((D, D), lambda i: (0, 0)),
        ],
        out_specs=[
            pl.BlockSpec((blk, D), lambda i: (i, 0)),
            pl.BlockSpec((blk, D), lambda i: (i, 0)),
        ],
        out_shape=[
            jax.ShapeDtypeStruct((N, D), _F32),
            jax.ShapeDtypeStruct((N, D), _F32),
        ],
    )(nf, gamma, beta, w1a, w1b)


def _edge_body(pi_ref, qj_ref, ef_ref, w1c_ref, b1_ref, w2_ref, b2_ref, out_ref):
    u = (pi_ref[...] + qj_ref[...]
         + jnp.dot(ef_ref[...], w1c_ref[...], preferred_element_type=_F32)
         + b1_ref[...][None, :])
    u = u * jax.nn.sigmoid(u)
    m = jnp.dot(u, w2_ref[...], preferred_element_type=_F32) + b2_ref[...][None, :]
    out_ref[...] = m * jax.nn.sigmoid(m)


def _edge_mlp(pi, qj, ef, w1c, b1, w2, b2):
    blk = 2000
    grid = (E // blk,)
    return pl.pallas_call(
        _edge_body,
        grid=grid,
        in_specs=[
            pl.BlockSpec((blk, D), lambda i: (i, 0)),
            pl.BlockSpec((blk, D), lambda i: (i, 0)),
            pl.BlockSpec((blk, ED), lambda i: (i, 0)),
            pl.BlockSpec((ED, D), lambda i: (0, 0)),
            pl.BlockSpec((D,), lambda i: (0,)),
            pl.BlockSpec((D, D), lambda i: (0, 0)),
            pl.BlockSpec((D,), lambda i: (0,)),
        ],
        out_specs=pl.BlockSpec((blk, D), lambda i: (i, 0)),
        out_shape=jax.ShapeDtypeStruct((E, D), _F32),
    )(pi, qj, ef, w1c, b1, w2, b2)


def _agg_body(nf_ref, ms_ref, cnt_ref, a1a_ref, a1b_ref, c1_ref, a2_ref, c2_ref,
              out_ref):
    nf = nf_ref[...]
    msg = ms_ref[...] / jnp.maximum(cnt_ref[...], 1.0)
    a = (jnp.dot(nf, a1a_ref[...], preferred_element_type=_F32)
         + jnp.dot(msg, a1b_ref[...], preferred_element_type=_F32)
         + c1_ref[...][None, :])
    a = a * jax.nn.sigmoid(a)
    a = jnp.dot(a, a2_ref[...], preferred_element_type=_F32) + c2_ref[...][None, :]
    out_ref[...] = nf + a * jax.nn.sigmoid(a)


def _agg(nf, msum, cnt, a1a, a1b, c1, a2, c2):
    blk = 1000
    grid = (N // blk,)
    return pl.pallas_call(
        _agg_body,
        grid=grid,
        in_specs=[
            pl.BlockSpec((blk, D), lambda i: (i, 0)),
            pl.BlockSpec((blk, D), lambda i: (i, 0)),
            pl.BlockSpec((blk, 1), lambda i: (i, 0)),
            pl.BlockSpec((D, D), lambda i: (0, 0)),
            pl.BlockSpec((D, D), lambda i: (0, 0)),
            pl.BlockSpec((D,), lambda i: (0,)),
            pl.BlockSpec((D, D), lambda i: (0, 0)),
            pl.BlockSpec((D,), lambda i: (0,)),
        ],
        out_specs=pl.BlockSpec((blk, D), lambda i: (i, 0)),
        out_shape=jax.ShapeDtypeStruct((N, D), _F32),
    )(nf, msum, cnt, a1a, a1b, c1, a2, c2)


def kernel(node_features, edge_features, edge_index, edge2graph,
           W1, b1, W2, b2, A1, c1, A2, c2, gamma, beta):
    del edge2graph
    w1a = W1[:D]
    w1b = W1[D:2 * D]
    w1c = W1[2 * D:]
    a1a = A1[:D]
    a1b = A1[D:]

    P, Q = _prep(node_features, gamma, beta, w1a, w1b)

    pi = jnp.take(P, edge_index[0], axis=0)
    qj = jnp.take(Q, edge_index[1], axis=0)

    mij = _edge_mlp(pi, qj, edge_features, w1c, b1, W2, b2)

    seg = edge_index[0]
    msum = jax.ops.segment_sum(mij, seg, num_segments=N)
    cnt = jax.ops.segment_sum(jnp.ones((E, 1), dtype=_F32), seg, num_segments=N)

    return _agg(node_features, msum, cnt, a1a, a1b, c1, A2, c2)


# speedup vs baseline: 2.8170x; 2.8170x over previous
import functools

import jax
import jax.numpy as jnp
from jax import lax
from jax.experimental import pallas as pl
from jax.experimental.pallas import tpu as pltpu
from jax.experimental.pallas import tpu_sc as plsc

N = 10000
E = 320000
D = 128
ED = 16

_F32 = jnp.float32

_NC = 2
_NS = 16
_NW = _NC * _NS
_CB = 128
_CHUNKS = E // _CB
_NPAD = 10240


def _sc_gather(P, Q, ei0, ei1):
    mesh = plsc.VectorSubcoreMesh(core_axis_name="c", subcore_axis_name="s")

    @functools.partial(
        pl.kernel,
        out_type=[jax.ShapeDtypeStruct((E, D), _F32),
                  jax.ShapeDtypeStruct((E, D), _F32)],
        mesh=mesh,
        scratch_types=[
            pltpu.VMEM((_CB,), jnp.int32),
            pltpu.VMEM((_CB,), jnp.int32),
            pltpu.VMEM((_CB, D), _F32),
            pltpu.VMEM((_CB, D), _F32),
            pltpu.SemaphoreType.DMA,
        ],
    )
    def k(p_hbm, q_hbm, ei0_hbm, ei1_hbm, pi_hbm, qj_hbm, ii_v, jj_v, pr_v, qr_v, sem):
        wid = lax.axis_index("s") * _NC + lax.axis_index("c")
        nk = jnp.where(wid < _CHUNKS % _NW, _CHUNKS // _NW + 1, _CHUNKS // _NW)

        def body(kk, carry):
            off = (kk * _NW + wid) * _CB
            pltpu.sync_copy(ei0_hbm.at[pl.ds(off, _CB)], ii_v)
            pltpu.sync_copy(ei1_hbm.at[pl.ds(off, _CB)], jj_v)
            pltpu.async_copy(p_hbm.at[ii_v], pr_v, sem).wait()
            pltpu.async_copy(q_hbm.at[jj_v], qr_v, sem).wait()
            pltpu.sync_copy(pr_v, pi_hbm.at[pl.ds(off, _CB)])
            pltpu.sync_copy(qr_v, qj_hbm.at[pl.ds(off, _CB)])
            return carry

        lax.fori_loop(0, nk, body, 0)

    return k(P, Q, ei0, ei1)


_NHALF = _NPAD // 2
_NACC = _NHALF + _CB
_ZCH = _NACC // _CB
_DCH = _NHALF // _CB


def _sc_scatter(mij, lcat, zd, z1, o1):
    mesh = plsc.VectorSubcoreMesh(core_axis_name="c", subcore_axis_name="s")

    @functools.partial(
        pl.kernel,
        out_type=[jax.ShapeDtypeStruct((_NPAD, D), _F32),
                  jax.ShapeDtypeStruct((_NPAD,), _F32)],
        mesh=mesh,
        scratch_types=[
            pltpu.VMEM((_CB, D), _F32),
            pltpu.VMEM((_CB,), jnp.int32),
            pltpu.VMEM((_CB,), _F32),
            pltpu.VMEM_SHARED((_NACC, D), _F32),
            pltpu.VMEM_SHARED((_NACC,), _F32),
        ],
    )
    def k(mij_hbm, lcat_hbm, zd_hbm, z1_hbm, o1_hbm, acc_hbm, cnt_hbm,
          mb_v, idx_v, ones_v, acc_sh, cnt_sh):
        cid = lax.axis_index("c")
        sid = lax.axis_index("s")
        base = cid * _NHALF

        pltpu.sync_copy(o1_hbm, ones_v)

        nz = jnp.where(sid < _ZCH % _NS, _ZCH // _NS + 1, _ZCH // _NS)

        def zero_acc(kk, carry):
            row = (kk * _NS + sid) * _CB
            pltpu.sync_copy(zd_hbm, acc_sh.at[pl.ds(row, _CB)])
            pltpu.sync_copy(z1_hbm, cnt_sh.at[pl.ds(row, _CB)])
            return carry

        lax.fori_loop(0, nz, zero_acc, 0)
        plsc.subcore_barrier()

        nk = jnp.where(sid < _CHUNKS % _NS, _CHUNKS // _NS + 1, _CHUNKS // _NS)

        def body(kk, carry):
            off = (kk * _NS + sid) * _CB
            pltpu.sync_copy(lcat_hbm.at[pl.ds(cid * E + off, _CB)], idx_v)
            pltpu.sync_copy(mij_hbm.at[pl.ds(off, _CB)], mb_v)
            pltpu.sync_copy(mb_v, acc_sh.at[idx_v], add=True)
            pltpu.sync_copy(ones_v, cnt_sh.at[idx_v], add=True)
            return carry

        lax.fori_loop(0, nk, body, 0)
        plsc.subcore_barrier()

        nd = jnp.where(sid < _DCH % _NS, _DCH // _NS + 1, _DCH // _NS)

        def dump(kk, carry):
            row = (kk * _NS + sid) * _CB
            pltpu.sync_copy(acc_sh.at[pl.ds(row, _CB)], acc_hbm.at[pl.ds(base + row, _CB)])
            pltpu.sync_copy(cnt_sh.at[pl.ds(row, _CB)], cnt_hbm.at[pl.ds(base + row, _CB)])
            return carry

        lax.fori_loop(0, nd, dump, 0)

    return k(mij, lcat, zd, z1, o1)


def _prep_body(nf_ref, g_ref, b_ref, wa_ref, wb_ref, p_ref, q_ref):
    x = nf_ref[...]
    mu = jnp.mean(x, axis=1, keepdims=True)
    var = jnp.mean((x - mu) ** 2, axis=1, keepdims=True)
    h = (x - mu) * lax.rsqrt(var + 1e-5) * g_ref[...][None, :] + b_ref[...][None, :]
    p_ref[...] = jnp.dot(h, wa_ref[...], preferred_element_type=_F32)
    q_ref[...] = jnp.dot(h, wb_ref[...], preferred_element_type=_F32)


def _prep(nf, gamma, beta, w1a, w1b):
    blk = 1000
    grid = (N // blk,)
    return pl.pallas_call(
        _prep_body,
        grid=grid,
        in_specs=[
            pl.BlockSpec((blk, D), lambda i: (i, 0)),
            pl.BlockSpec((D,), lambda i: (0,)),
            pl.BlockSpec((D,), lambda i: (0,)),
            pl.BlockSpec((D, D), lambda i: (0, 0)),
            pl.BlockSpec((D, D), lambda i: (0, 0)),
        ],
        out_specs=[
            pl.BlockSpec((blk, D), lambda i: (i, 0)),
            pl.BlockSpec((blk, D), lambda i: (i, 0)),
        ],
        out_shape=[
            jax.ShapeDtypeStruct((N, D), _F32),
            jax.ShapeDtypeStruct((N, D), _F32),
        ],
    )(nf, gamma, beta, w1a, w1b)


def _edge_body(pi_ref, qj_ref, ef_ref, w1c_ref, b1_ref, w2_ref, b2_ref, out_ref):
    u = (pi_ref[...] + qj_ref[...]
         + jnp.dot(ef_ref[...], w1c_ref[...], preferred_element_type=_F32)
         + b1_ref[...][None, :])
    u = u * jax.nn.sigmoid(u)
    m = jnp.dot(u, w2_ref[...], preferred_element_type=_F32) + b2_ref[...][None, :]
    out_ref[...] = m * jax.nn.sigmoid(m)


def _edge_mlp(pi, qj, ef, w1c, b1, w2, b2):
    blk = 2000
    grid = (E // blk,)
    return pl.pallas_call(
        _edge_body,
        grid=grid,
        in_specs=[
            pl.BlockSpec((blk, D), lambda i: (i, 0)),
            pl.BlockSpec((blk, D), lambda i: (i, 0)),
            pl.BlockSpec((blk, ED), lambda i: (i, 0)),
            pl.BlockSpec((ED, D), lambda i: (0, 0)),
            pl.BlockSpec((D,), lambda i: (0,)),
            pl.BlockSpec((D, D), lambda i: (0, 0)),
            pl.BlockSpec((D,), lambda i: (0,)),
        ],
        out_specs=pl.BlockSpec((blk, D), lambda i: (i, 0)),
        out_shape=jax.ShapeDtypeStruct((E, D), _F32),
    )(pi, qj, ef, w1c, b1, w2, b2)


def _agg_body(nf_ref, ms_ref, cnt_ref, a1a_ref, a1b_ref, c1_ref, a2_ref, c2_ref,
              out_ref):
    nf = nf_ref[...]
    msg = ms_ref[...] / jnp.maximum(cnt_ref[...], 1.0)
    a = (jnp.dot(nf, a1a_ref[...], preferred_element_type=_F32)
         + jnp.dot(msg, a1b_ref[...], preferred_element_type=_F32)
         + c1_ref[...][None, :])
    a = a * jax.nn.sigmoid(a)
    a = jnp.dot(a, a2_ref[...], preferred_element_type=_F32) + c2_ref[...][None, :]
    out_ref[...] = nf + a * jax.nn.sigmoid(a)


def _agg(nf, msum, cnt, a1a, a1b, c1, a2, c2):
    blk = 1000
    grid = (N // blk,)
    return pl.pallas_call(
        _agg_body,
        grid=grid,
        in_specs=[
            pl.BlockSpec((blk, D), lambda i: (i, 0)),
            pl.BlockSpec((blk, D), lambda i: (i, 0)),
            pl.BlockSpec((blk, 1), lambda i: (i, 0)),
            pl.BlockSpec((D, D), lambda i: (0, 0)),
            pl.BlockSpec((D, D), lambda i: (0, 0)),
            pl.BlockSpec((D,), lambda i: (0,)),
            pl.BlockSpec((D, D), lambda i: (0, 0)),
            pl.BlockSpec((D,), lambda i: (0,)),
        ],
        out_specs=pl.BlockSpec((blk, D), lambda i: (i, 0)),
        out_shape=jax.ShapeDtypeStruct((N, D), _F32),
    )(nf, msum, cnt, a1a, a1b, c1, a2, c2)


def kernel(node_features, edge_features, edge_index, edge2graph,
           W1, b1, W2, b2, A1, c1, A2, c2, gamma, beta):
    del edge2graph
    w1a = W1[:D]
    w1b = W1[D:2 * D]
    w1c = W1[2 * D:]
    a1a = A1[:D]
    a1b = A1[D:]

    P, Q = _prep(node_features, gamma, beta, w1a, w1b)

    ei0 = edge_index[0]
    ei1 = edge_index[1]
    pi, qj = _sc_gather(P, Q, ei0, ei1)

    mij = _edge_mlp(pi, qj, edge_features, w1c, b1, W2, b2)

    l0 = jnp.where(ei0 < _NHALF, ei0, _NHALF)
    l1 = jnp.where(ei0 >= _NHALF, ei0 - _NHALF, _NHALF)
    lcat = jnp.concatenate([l0, l1])
    zd = jnp.zeros((_CB, D), _F32)
    z1 = jnp.zeros((_CB,), _F32)
    o1 = jnp.ones((_CB,), _F32)
    acc, cnt = _sc_scatter(mij, lcat, zd, z1, o1)

    return _agg(node_features, acc, cnt.reshape(_NPAD, 1), a1a, a1b, c1, A2, c2)


# speedup vs baseline: 3.3692x; 1.1960x over previous
import functools

import jax
import jax.numpy as jnp
from jax import lax
from jax.experimental import pallas as pl
from jax.experimental.pallas import tpu as pltpu
from jax.experimental.pallas import tpu_sc as plsc

N = 10000
E = 320000
D = 128
ED = 16

_F32 = jnp.float32

_NC = 2
_NS = 16
_NW = _NC * _NS
_CB = 128
_CHUNKS = E // _CB
_NPAD = 10240


def _sc_gather(P, Q, ei0, ei1):
    mesh = plsc.VectorSubcoreMesh(core_axis_name="c", subcore_axis_name="s")

    @functools.partial(
        pl.kernel,
        out_type=[jax.ShapeDtypeStruct((E, D), _F32),
                  jax.ShapeDtypeStruct((E, D), _F32)],
        mesh=mesh,
        scratch_types=[
            pltpu.VMEM((_CB,), jnp.int32),
            pltpu.VMEM((_CB,), jnp.int32),
            pltpu.VMEM((_CB,), jnp.int32),
            pltpu.VMEM((_CB,), jnp.int32),
            pltpu.VMEM((_CB, D), _F32),
            pltpu.VMEM((_CB, D), _F32),
            pltpu.VMEM((_CB, D), _F32),
            pltpu.VMEM((_CB, D), _F32),
            pltpu.SemaphoreType.DMA,
            pltpu.SemaphoreType.DMA,
        ],
    )
    def k(p_hbm, q_hbm, ei0_hbm, ei1_hbm, pi_hbm, qj_hbm,
          ii0, jj0, ii1, jj1, pr0, qr0, pr1, qr1, sem_i, sem_g):
        wid = lax.axis_index("s") * _NC + lax.axis_index("c")
        extra = jnp.where(wid < _CHUNKS % _NW, 1, 0)
        nks = _CHUNKS // _NW
        nk = nks + extra

        def off_(c):
            return (c * _NW + wid) * _CB

        def load_idx(c, ii, jj):
            a = pltpu.async_copy(ei0_hbm.at[pl.ds(off_(c), _CB)], ii, sem_i)
            b = pltpu.async_copy(ei1_hbm.at[pl.ds(off_(c), _CB)], jj, sem_i)
            return a, b

        def drain_idx(ii, jj):
            pltpu.make_async_copy(ei0_hbm.at[pl.ds(0, _CB)], ii, sem_i).wait()
            pltpu.make_async_copy(ei1_hbm.at[pl.ds(0, _CB)], jj, sem_i).wait()

        def fire_g(ii, jj, pr, qr):
            pltpu.async_copy(p_hbm.at[ii], pr, sem_g)
            pltpu.async_copy(q_hbm.at[jj], qr, sem_g)

        def drain_g(ii, jj, pr, qr):
            pltpu.make_async_copy(p_hbm.at[ii], pr, sem_g).wait()
            pltpu.make_async_copy(q_hbm.at[jj], qr, sem_g).wait()

        def write(c, pr, qr):
            pltpu.sync_copy(pr, pi_hbm.at[pl.ds(off_(c), _CB)])
            pltpu.sync_copy(qr, qj_hbm.at[pl.ds(off_(c), _CB)])

        pltpu.sync_copy(ei0_hbm.at[pl.ds(off_(0), _CB)], ii0)
        pltpu.sync_copy(ei1_hbm.at[pl.ds(off_(0), _CB)], jj0)
        fire_g(ii0, jj0, pr0, qr0)

        def body(kk, carry):
            a = 2 * kk
            b = 2 * kk + 1
            c = jnp.minimum(2 * kk + 2, nk - 1)
            load_idx(b, ii1, jj1)
            drain_g(ii0, jj0, pr0, qr0)
            drain_idx(ii1, jj1)
            fire_g(ii1, jj1, pr1, qr1)
            write(a, pr0, qr0)
            load_idx(c, ii0, jj0)
            drain_g(ii1, jj1, pr1, qr1)
            drain_idx(ii0, jj0)
            fire_g(ii0, jj0, pr0, qr0)
            write(b, pr1, qr1)
            return carry

        lax.fori_loop(0, nks // 2, body, 0)

        drain_g(ii0, jj0, pr0, qr0)

        def ebody(_, carry):
            write(nks, pr0, qr0)
            return carry

        lax.fori_loop(0, extra, ebody, 0)

    return k(P, Q, ei0, ei1)


_NHALF = _NPAD // 2
_NACC = _NHALF + _CB
_ZCH = _NACC // _CB
_DCH = _NHALF // _CB


def _sc_scatter(mij, lcat, zd, z1, o1):
    mesh = plsc.VectorSubcoreMesh(core_axis_name="c", subcore_axis_name="s")

    @functools.partial(
        pl.kernel,
        out_type=[jax.ShapeDtypeStruct((_NPAD, D), _F32),
                  jax.ShapeDtypeStruct((_NPAD,), _F32)],
        mesh=mesh,
        scratch_types=[
            pltpu.VMEM((_CB, D), _F32),
            pltpu.VMEM((_CB, D), _F32),
            pltpu.VMEM((_CB,), jnp.int32),
            pltpu.VMEM((_CB,), jnp.int32),
            pltpu.VMEM((_CB,), _F32),
            pltpu.VMEM_SHARED((_NACC, D), _F32),
            pltpu.VMEM_SHARED((_NACC,), _F32),
            pltpu.SemaphoreType.DMA,
        ],
    )
    def k(mij_hbm, lcat_hbm, zd_hbm, z1_hbm, o1_hbm, acc_hbm, cnt_hbm,
          mb0, mb1, idx0, idx1, ones_v, acc_sh, cnt_sh, sem_l):
        cid = lax.axis_index("c")
        sid = lax.axis_index("s")
        base = cid * _NHALF

        pltpu.sync_copy(o1_hbm, ones_v)

        nz = jnp.where(sid < _ZCH % _NS, _ZCH // _NS + 1, _ZCH // _NS)

        def zero_acc(kk, carry):
            row = (kk * _NS + sid) * _CB
            pltpu.sync_copy(zd_hbm, acc_sh.at[pl.ds(row, _CB)])
            pltpu.sync_copy(z1_hbm, cnt_sh.at[pl.ds(row, _CB)])
            return carry

        lax.fori_loop(0, nz, zero_acc, 0)
        plsc.subcore_barrier()

        extra = jnp.where(sid < _CHUNKS % _NS, 1, 0)
        nks = _CHUNKS // _NS
        nk = nks + extra

        def off_(c):
            return (c * _NS + sid) * _CB

        def load(c, idx, mb):
            pltpu.async_copy(lcat_hbm.at[pl.ds(cid * E + off_(c), _CB)], idx, sem_l)
            pltpu.async_copy(mij_hbm.at[pl.ds(off_(c), _CB)], mb, sem_l)

        def drain_load(idx, mb):
            pltpu.make_async_copy(lcat_hbm.at[pl.ds(0, _CB)], idx, sem_l).wait()
            pltpu.make_async_copy(mij_hbm.at[pl.ds(0, _CB)], mb, sem_l).wait()

        def scat(idx, mb):
            pltpu.sync_copy(mb, acc_sh.at[idx], add=True)
            pltpu.sync_copy(ones_v, cnt_sh.at[idx], add=True)

        load(0, idx0, mb0)

        def body(kk, carry):
            b = 2 * kk + 1
            c = jnp.minimum(2 * kk + 2, nk - 1)
            load(b, idx1, mb1)
            drain_load(idx0, mb0)
            scat(idx0, mb0)
            load(c, idx0, mb0)
            drain_load(idx1, mb1)
            scat(idx1, mb1)
            return carry

        lax.fori_loop(0, nks // 2, body, 0)

        drain_load(idx0, mb0)

        def ebody(_, carry):
            scat(idx0, mb0)
            return carry

        lax.fori_loop(0, extra, ebody, 0)
        plsc.subcore_barrier()

        nd = jnp.where(sid < _DCH % _NS, _DCH // _NS + 1, _DCH // _NS)

        def dump(kk, carry):
            row = (kk * _NS + sid) * _CB
            pltpu.sync_copy(acc_sh.at[pl.ds(row, _CB)], acc_hbm.at[pl.ds(base + row, _CB)])
            pltpu.sync_copy(cnt_sh.at[pl.ds(row, _CB)], cnt_hbm.at[pl.ds(base + row, _CB)])
            return carry

        lax.fori_loop(0, nd, dump, 0)

    return k(mij, lcat, zd, z1, o1)


def _prep_body(nf_ref, g_ref, b_ref, wa_ref, wb_ref, p_ref, q_ref):
    x = nf_ref[...]
    mu = jnp.mean(x, axis=1, keepdims=True)
    var = jnp.mean((x - mu) ** 2, axis=1, keepdims=True)
    h = (x - mu) * lax.rsqrt(var + 1e-5) * g_ref[...][None, :] + b_ref[...][None, :]
    p_ref[...] = jnp.dot(h, wa_ref[...], preferred_element_type=_F32)
    q_ref[...] = jnp.dot(h, wb_ref[...], preferred_element_type=_F32)


def _prep(nf, gamma, beta, w1a, w1b):
    blk = 1000
    grid = (N // blk,)
    return pl.pallas_call(
        _prep_body,
        grid=grid,
        in_specs=[
            pl.BlockSpec((blk, D), lambda i: (i, 0)),
            pl.BlockSpec((D,), lambda i: (0,)),
            pl.BlockSpec((D,), lambda i: (0,)),
            pl.BlockSpec((D, D), lambda i: (0, 0)),
            pl.BlockSpec((D, D), lambda i: (0, 0)),
        ],
        out_specs=[
            pl.BlockSpec((blk, D), lambda i: (i, 0)),
            pl.BlockSpec((blk, D), lambda i: (i, 0)),
        ],
        out_shape=[
            jax.ShapeDtypeStruct((N, D), _F32),
            jax.ShapeDtypeStruct((N, D), _F32),
        ],
    )(nf, gamma, beta, w1a, w1b)


def _edge_body(pi_ref, qj_ref, ef_ref, w1c_ref, b1_ref, w2_ref, b2_ref, out_ref):
    u = (pi_ref[...] + qj_ref[...]
         + jnp.dot(ef_ref[...], w1c_ref[...], preferred_element_type=_F32)
         + b1_ref[...][None, :])
    u = u * jax.nn.sigmoid(u)
    m = jnp.dot(u, w2_ref[...], preferred_element_type=_F32) + b2_ref[...][None, :]
    out_ref[...] = m * jax.nn.sigmoid(m)


def _edge_mlp(pi, qj, ef, w1c, b1, w2, b2):
    blk = 2000
    grid = (E // blk,)
    return pl.pallas_call(
        _edge_body,
        grid=grid,
        in_specs=[
            pl.BlockSpec((blk, D), lambda i: (i, 0)),
            pl.BlockSpec((blk, D), lambda i: (i, 0)),
            pl.BlockSpec((blk, ED), lambda i: (i, 0)),
            pl.BlockSpec((ED, D), lambda i: (0, 0)),
            pl.BlockSpec((D,), lambda i: (0,)),
            pl.BlockSpec((D, D), lambda i: (0, 0)),
            pl.BlockSpec((D,), lambda i: (0,)),
        ],
        out_specs=pl.BlockSpec((blk, D), lambda i: (i, 0)),
        out_shape=jax.ShapeDtypeStruct((E, D), _F32),
    )(pi, qj, ef, w1c, b1, w2, b2)


def _agg_body(nf_ref, ms_ref, cnt_ref, a1a_ref, a1b_ref, c1_ref, a2_ref, c2_ref,
              out_ref):
    nf = nf_ref[...]
    msg = ms_ref[...] / jnp.maximum(cnt_ref[...], 1.0)
    a = (jnp.dot(nf, a1a_ref[...], preferred_element_type=_F32)
         + jnp.dot(msg, a1b_ref[...], preferred_element_type=_F32)
         + c1_ref[...][None, :])
    a = a * jax.nn.sigmoid(a)
    a = jnp.dot(a, a2_ref[...], preferred_element_type=_F32) + c2_ref[...][None, :]
    out_ref[...] = nf + a * jax.nn.sigmoid(a)


def _agg(nf, msum, cnt, a1a, a1b, c1, a2, c2):
    blk = 1000
    grid = (N // blk,)
    return pl.pallas_call(
        _agg_body,
        grid=grid,
        in_specs=[
            pl.BlockSpec((blk, D), lambda i: (i, 0)),
            pl.BlockSpec((blk, D), lambda i: (i, 0)),
            pl.BlockSpec((blk, 1), lambda i: (i, 0)),
            pl.BlockSpec((D, D), lambda i: (0, 0)),
            pl.BlockSpec((D, D), lambda i: (0, 0)),
            pl.BlockSpec((D,), lambda i: (0,)),
            pl.BlockSpec((D, D), lambda i: (0, 0)),
            pl.BlockSpec((D,), lambda i: (0,)),
        ],
        out_specs=pl.BlockSpec((blk, D), lambda i: (i, 0)),
        out_shape=jax.ShapeDtypeStruct((N, D), _F32),
    )(nf, msum, cnt, a1a, a1b, c1, a2, c2)


def kernel(node_features, edge_features, edge_index, edge2graph,
           W1, b1, W2, b2, A1, c1, A2, c2, gamma, beta):
    del edge2graph
    w1a = W1[:D]
    w1b = W1[D:2 * D]
    w1c = W1[2 * D:]
    a1a = A1[:D]
    a1b = A1[D:]

    P, Q = _prep(node_features, gamma, beta, w1a, w1b)

    ei0 = edge_index[0]
    ei1 = edge_index[1]
    pi, qj = _sc_gather(P, Q, ei0, ei1)

    mij = _edge_mlp(pi, qj, edge_features, w1c, b1, W2, b2)

    l0 = jnp.where(ei0 < _NHALF, ei0, _NHALF)
    l1 = jnp.where(ei0 >= _NHALF, ei0 - _NHALF, _NHALF)
    lcat = jnp.concatenate([l0, l1])
    zd = jnp.zeros((_CB, D), _F32)
    z1 = jnp.zeros((_CB,), _F32)
    o1 = jnp.ones((_CB,), _F32)
    acc, cnt = _sc_scatter(mij, lcat, zd, z1, o1)

    return _agg(node_features, acc, cnt.reshape(_NPAD, 1), a1a, a1b, c1, A2, c2)
